# baseline (speedup 1.0000x reference)
import jax
import jax.numpy as jnp
from jax import lax
from jax.experimental import pallas as pl
from jax.experimental.pallas import tpu as pltpu
from jax.experimental.pallas import tpu_sc as plsc

_N = 100000
_D = 64
_B = 16384
_MARGIN = 1.0
_BETA1 = 0.1

_NC = 2
_NS = 16
_NW = _NC * _NS
_FPW = _D // _NW
_L = 16
_CHUNK = 2048
_NCH = _B // _CHUNK
_UNROLL = 4


def _delta_body(et_hbm, idx_hbm, out_hbm, row_v, idx_v, part_v):
    cid = lax.axis_index("c")
    sid = lax.axis_index("s")
    wid = sid * _NC + cid

    for fi in range(_FPW):
        f = wid * _FPW + fi
        pltpu.sync_copy(et_hbm.at[f], row_v)
        for ch in range(_NCH):
            for t in range(3):
                pltpu.sync_copy(idx_hbm.at[t, pl.ds(ch * _CHUNK, _CHUNK)],
                                idx_v.at[t])

            def group(g4, carry):
                for u in range(_UNROLL):
                    g = g4 * _UNROLL + u
                    jv = idx_v[0, pl.ds(g * _L, _L)]
                    kv = idx_v[1, pl.ds(g * _L, _L)]
                    nv = idx_v[2, pl.ds(g * _L, _L)]
                    gj = plsc.load_gather(row_v, [jv])
                    gk = plsc.load_gather(row_v, [kv])
                    gn = plsc.load_gather(row_v, [nv])
                    d = gj - gn
                    m = gj + gn - gk - gk
                    c = d * m
                    off = ch * _CHUNK + g * _L
                    if fi == 0:
                        part_v[pl.ds(off, _L)] = c
                    else:
                        part_v[pl.ds(off, _L)] += c
                return carry

            lax.fori_loop(0, _CHUNK // _L // _UNROLL, group, jnp.int32(0))

    pltpu.sync_copy(part_v, out_hbm.at[wid])


def _delta(et_t, idx):
    mesh = plsc.VectorSubcoreMesh(core_axis_name="c", subcore_axis_name="s")
    return pl.kernel(
        _delta_body,
        out_type=jax.ShapeDtypeStruct((_NW, _B), jnp.float32),
        mesh=mesh,
        scratch_types=[
            pltpu.VMEM((_N,), jnp.float32),
            pltpu.VMEM((3, _CHUNK), jnp.int32),
            pltpu.VMEM((_B,), jnp.float32),
        ],
        compiler_params=pltpu.CompilerParams(use_tc_tiling_on_sc=False,
                                             needs_layout_passes=False),
    )(et_t, idx)


_SROWS = 8


def _smooth_body(e_ref, l_ref, out_ref):
    i = pl.program_id(0)
    d = e_ref[...] - l_ref[...]
    s = jnp.sum(d * d)

    @pl.when(i == 0)
    def _():
        out_ref[0, 0] = s

    @pl.when(i > 0)
    def _():
        out_ref[0, 0] += s


def _smooth(e2, l2):
    grid = e2.shape[0] // _SROWS
    return pl.pallas_call(
        _smooth_body,
        grid=(grid,),
        in_specs=[
            pl.BlockSpec((_SROWS, _N), lambda i: (i, 0)),
            pl.BlockSpec((_SROWS, _N), lambda i: (i, 0)),
        ],
        out_specs=pl.BlockSpec(memory_space=pltpu.SMEM),
        out_shape=jax.ShapeDtypeStruct((1, 1), jnp.float32),
    )(e2, l2)


def _fin_body(dp_ref, sm_ref, out_ref):
    d = jnp.sum(dp_ref[...], axis=0)
    h = jnp.maximum(d + _MARGIN, 0.0)
    out_ref[0, 0] = jnp.sum(h) + _BETA1 * (float(_B) * sm_ref[0, 0])


def _fin(dp, sm):
    return pl.pallas_call(
        _fin_body,
        in_specs=[
            pl.BlockSpec((_NW, _B), lambda: (0, 0)),
            pl.BlockSpec(memory_space=pltpu.SMEM),
        ],
        out_specs=pl.BlockSpec(memory_space=pltpu.SMEM),
        out_shape=jax.ShapeDtypeStruct((1, 1), jnp.float32),
    )(dp, sm)


@jax.jit
def kernel(embeddings, last_embeddings, triplets):
    idx = triplets.astype(jnp.int32).T
    dp = _delta(embeddings.T, idx)
    sm = _smooth(embeddings.T, last_embeddings.T)
    return _fin(dp, sm)[0, 0]


# speedup vs baseline: 1.2839x; 1.2839x over previous
import jax
import jax.numpy as jnp
from jax import lax
from jax.experimental import pallas as pl
from jax.experimental.pallas import tpu as pltpu
from jax.experimental.pallas import tpu_sc as plsc

_N = 100000
_D = 64
_B = 16384
_MARGIN = 1.0
_BETA1 = 0.1

_NC = 2
_NS = 16
_NW = _NC * _NS
_FPW = _D // _NW
_L = 16
_CHUNK = 2048
_NCH = _B // _CHUNK
_GPC = _CHUNK // _L


def _delta_body(et_hbm, idx_hbm, out_hbm, row_v, idx_v, part_v,
                rsem, sem0, sem1):
    cid = lax.axis_index("c")
    sid = lax.axis_index("s")
    wid = sid * _NC + cid
    sems = (sem0, sem1)

    def issue_idx(ch):
        buf = ch % 2
        return [
            pltpu.async_copy(idx_hbm.at[t, pl.ds(ch * _CHUNK, _CHUNK)],
                             idx_v.at[buf, t], sems[buf])
            for t in range(3)
        ]

    for fi in range(_FPW):
        f = wid * _FPW + fi
        rc = pltpu.async_copy(et_hbm.at[f], row_v, rsem)
        pend = issue_idx(0)
        rc.wait()
        for ch in range(_NCH):
            buf = ch % 2
            for c in pend:
                c.wait()
            if ch + 1 < _NCH:
                pend = issue_idx(ch + 1)

            @plsc.parallel_loop(0, _GPC, 1, unroll=8)
            def _group(g):
                jv = idx_v[buf, 0, pl.ds(g * _L, _L)]
                kv = idx_v[buf, 1, pl.ds(g * _L, _L)]
                nv = idx_v[buf, 2, pl.ds(g * _L, _L)]
                gj = plsc.load_gather(row_v, [jv])
                gk = plsc.load_gather(row_v, [kv])
                gn = plsc.load_gather(row_v, [nv])
                d = gj - gn
                m = gj + gn - gk - gk
                part_v[pl.ds(ch * _CHUNK + g * _L, _L)] = d * m

        pltpu.sync_copy(part_v, out_hbm.at[f])


def _delta(et_t, idx):
    mesh = plsc.VectorSubcoreMesh(core_axis_name="c", subcore_axis_name="s")
    return pl.kernel(
        _delta_body,
        out_type=jax.ShapeDtypeStruct((_D, _B), jnp.float32),
        mesh=mesh,
        scratch_types=[
            pltpu.VMEM((_N,), jnp.float32),
            pltpu.VMEM((2, 3, _CHUNK), jnp.int32),
            pltpu.VMEM((_B,), jnp.float32),
            pltpu.SemaphoreType.DMA,
            pltpu.SemaphoreType.DMA,
            pltpu.SemaphoreType.DMA,
        ],
        compiler_params=pltpu.CompilerParams(use_tc_tiling_on_sc=False,
                                             needs_layout_passes=False),
    )(et_t, idx)


_SROWS = 8


def _smooth_body(e_ref, l_ref, out_ref):
    i = pl.program_id(0)
    d = e_ref[...] - l_ref[...]
    s = jnp.sum(d * d)

    @pl.when(i == 0)
    def _():
        out_ref[0, 0] = s

    @pl.when(i > 0)
    def _():
        out_ref[0, 0] += s


def _smooth(e2, l2):
    grid = e2.shape[0] // _SROWS
    return pl.pallas_call(
        _smooth_body,
        grid=(grid,),
        in_specs=[
            pl.BlockSpec((_SROWS, _N), lambda i: (i, 0)),
            pl.BlockSpec((_SROWS, _N), lambda i: (i, 0)),
        ],
        out_specs=pl.BlockSpec(memory_space=pltpu.SMEM),
        out_shape=jax.ShapeDtypeStruct((1, 1), jnp.float32),
    )(e2, l2)


def _fin_body(dp_ref, sm_ref, out_ref):
    d = jnp.sum(dp_ref[...], axis=0)
    h = jnp.maximum(d + _MARGIN, 0.0)
    out_ref[0, 0] = jnp.sum(h) + _BETA1 * (float(_B) * sm_ref[0, 0])


def _fin(dp, sm):
    return pl.pallas_call(
        _fin_body,
        in_specs=[
            pl.BlockSpec((_D, _B), lambda: (0, 0)),
            pl.BlockSpec(memory_space=pltpu.SMEM),
        ],
        out_specs=pl.BlockSpec(memory_space=pltpu.SMEM),
        out_shape=jax.ShapeDtypeStruct((1, 1), jnp.float32),
    )(dp, sm)


@jax.jit
def kernel(embeddings, last_embeddings, triplets):
    idx = triplets.astype(jnp.int32).T
    dp = _delta(embeddings.T, idx)
    sm = _smooth(embeddings.T, last_embeddings.T)
    return _fin(dp, sm)[0, 0]
